# baseline (speedup 1.0000x reference)
import functools

import jax
import jax.numpy as jnp
from jax import lax
from jax.experimental import pallas as pl
from jax.experimental.pallas import tpu as pltpu
from jax.experimental.pallas import tpu_sc as plsc

_IDX_CHUNK = 128


def _sc_gather(table, ids):
    B = ids.shape[0]
    D = table.shape[1]
    info = plsc.get_sparse_core_info()
    NC, NS = info.num_cores, info.num_subcores
    NW = NC * NS
    b_per_w = B // NW
    assert B % (NW * _IDX_CHUNK) == 0
    n_chunks = b_per_w // _IDX_CHUNK
    ids3 = ids.reshape(NW, n_chunks, _IDX_CHUNK)
    mesh = plsc.VectorSubcoreMesh(core_axis_name="c", subcore_axis_name="s")

    @functools.partial(
        pl.kernel,
        mesh=mesh,
        out_type=jax.ShapeDtypeStruct((B, D), jnp.float32),
        scratch_types=[
            pltpu.VMEM((n_chunks, _IDX_CHUNK), jnp.int32),
            pltpu.VMEM((b_per_w, D), jnp.float32),
            pltpu.SemaphoreType.DMA,
        ],
        compiler_params=pltpu.CompilerParams(use_tc_tiling_on_sc=False),
    )
    def gather_kernel(table_hbm, ids_hbm, out_hbm, idx_v, rows_v, sem):
        wid = lax.axis_index("s") * NC + lax.axis_index("c")
        pltpu.sync_copy(ids_hbm.at[wid], idx_v)
        copies = []
        for j in range(n_chunks):
            copies.append(
                pltpu.async_copy(
                    table_hbm.at[idx_v.at[j]],
                    rows_v.at[pl.ds(j * _IDX_CHUNK, _IDX_CHUNK)],
                    sem,
                )
            )
        for c in copies:
            c.wait()
        pltpu.sync_copy(rows_v, out_hbm.at[pl.ds(wid * b_per_w, b_per_w)])

    return gather_kernel(table, ids3)


def _tc_combine(uf, tv, w1, w2, b2d):
    B, D = uf.shape
    bB = 2048

    def body(uf_ref, tv_ref, w1_ref, w2_ref, b_ref, o_ref):
        acc = jnp.dot(uf_ref[...], w1_ref[...], preferred_element_type=jnp.float32)
        acc += jnp.dot(tv_ref[...], w2_ref[...], preferred_element_type=jnp.float32)
        o_ref[...] = acc + b_ref[...]

    return pl.pallas_call(
        body,
        grid=(B // bB,),
        in_specs=[
            pl.BlockSpec((bB, D), lambda i: (i, 0)),
            pl.BlockSpec((bB, D), lambda i: (i, 0)),
            pl.BlockSpec((D, D), lambda i: (0, 0)),
            pl.BlockSpec((D, D), lambda i: (0, 0)),
            pl.BlockSpec((1, D), lambda i: (0, 0)),
        ],
        out_specs=pl.BlockSpec((bB, D), lambda i: (i, 0)),
        out_shape=jax.ShapeDtypeStruct((B, D), jnp.float32),
        compiler_params=pltpu.CompilerParams(
            dimension_semantics=("arbitrary",),
        ),
    )(uf, tv, w1, w2, b2d)


@jax.jit
def kernel(content, table, W, b):
    D = table.shape[1]
    ids = content[:, 0].astype(jnp.int32)
    tv = content[:, 1:]
    uf = _sc_gather(table, ids)
    w1 = W[:, :D].T
    w2 = W[:, D:].T
    return _tc_combine(uf, tv, w1, w2, b.reshape(1, D))


# speedup vs baseline: 1.6829x; 1.6829x over previous
import functools

import jax
import jax.numpy as jnp
from jax import lax
from jax.experimental import pallas as pl
from jax.experimental.pallas import tpu as pltpu
from jax.experimental.pallas import tpu_sc as plsc


def _sc_gather(table, ids):
    V, D = table.shape
    B = ids.shape[0]
    info = plsc.get_sparse_core_info()
    NC, NS = info.num_cores, info.num_subcores
    NW = NC * NS
    b_per_w = B // NW
    assert B % NW == 0
    ids2 = ids.reshape(NW, b_per_w)
    mesh = plsc.VectorSubcoreMesh(core_axis_name="c", subcore_axis_name="s")

    @functools.partial(
        pl.kernel,
        mesh=mesh,
        out_type=jax.ShapeDtypeStruct((B, D), jnp.float32),
        scratch_types=[
            pltpu.VMEM((b_per_w,), jnp.int32),
            pltpu.VMEM((b_per_w, D), jnp.float32),
            pltpu.SemaphoreType.DMA,
        ],
        compiler_params=pltpu.CompilerParams(needs_layout_passes=False),
    )
    def gather_kernel(table_hbm, ids_hbm, out_hbm, ids_v, rows_v, sem):
        wid = lax.axis_index("s") * NC + lax.axis_index("c")
        pltpu.sync_copy(ids_hbm.at[wid], ids_v)

        def fire(g, _):
            vec = ids_v[pl.ds(g * 16, 16)]
            for l in range(16):
                rid = vec[l]
                pltpu.async_copy(
                    table_hbm.at[pl.ds(rid, 1)],
                    rows_v.at[pl.ds(g * 16 + l, 1)],
                    sem,
                )
            return 0

        lax.fori_loop(0, b_per_w // 16, fire, 0)
        pltpu.make_async_copy(
            table_hbm.at[pl.ds(0, b_per_w)], rows_v, sem
        ).wait()
        pltpu.sync_copy(rows_v, out_hbm.at[pl.ds(wid * b_per_w, b_per_w)])

    return gather_kernel(table, ids2)


def _tc_combine(uf, tv, w1, w2, b2d):
    B, D = uf.shape
    bB = 2048

    def body(uf_ref, tv_ref, w1_ref, w2_ref, b_ref, o_ref):
        acc = jnp.dot(uf_ref[...], w1_ref[...], preferred_element_type=jnp.float32)
        acc += jnp.dot(tv_ref[...], w2_ref[...], preferred_element_type=jnp.float32)
        o_ref[...] = acc + b_ref[...]

    return pl.pallas_call(
        body,
        grid=(B // bB,),
        in_specs=[
            pl.BlockSpec((bB, D), lambda i: (i, 0)),
            pl.BlockSpec((bB, D), lambda i: (i, 0)),
            pl.BlockSpec((D, D), lambda i: (0, 0)),
            pl.BlockSpec((D, D), lambda i: (0, 0)),
            pl.BlockSpec((1, D), lambda i: (0, 0)),
        ],
        out_specs=pl.BlockSpec((bB, D), lambda i: (i, 0)),
        out_shape=jax.ShapeDtypeStruct((B, D), jnp.float32),
        compiler_params=pltpu.CompilerParams(
            dimension_semantics=("arbitrary",),
        ),
    )(uf, tv, w1, w2, b2d)


@jax.jit
def kernel(content, table, W, b):
    D = table.shape[1]
    ids = content[:, 0].astype(jnp.int32)
    tv = content[:, 1:]
    uf = _sc_gather(table, ids)
    w1 = W[:, :D].T
    w2 = W[:, D:].T
    return _tc_combine(uf, tv, w1, w2, b.reshape(1, D))


# speedup vs baseline: 3.8769x; 2.3038x over previous
import functools

import jax
import jax.numpy as jnp
from jax import lax
from jax.experimental import pallas as pl
from jax.experimental.pallas import tpu as pltpu
from jax.experimental.pallas import tpu_sc as plsc

_CH = 512
_WAVE = 32


def _sc_gather_scan(tableT, ids):
    D, V = tableT.shape
    B = ids.shape[0]
    info = plsc.get_sparse_core_info()
    NC, NS = info.num_cores, info.num_subcores
    NW = NC * NS
    n_full = V // _CH
    tail = V - n_full * _CH
    K = (n_full + NW - 1) // NW
    tail_owner = n_full % NW
    nvec = B // 16
    mesh = plsc.VectorSubcoreMesh(core_axis_name="c", subcore_axis_name="s")

    @functools.partial(
        pl.kernel,
        mesh=mesh,
        out_type=jax.ShapeDtypeStruct((B, D), jnp.float32),
        scratch_types=[
            pltpu.VMEM((B,), jnp.int32),
            pltpu.VMEM((B,), jnp.int32),
            pltpu.VMEM((B,), jnp.int32),
            pltpu.VMEM((D, _CH), jnp.float32),
            pltpu.VMEM((D, _CH), jnp.float32),
            pltpu.VMEM((_WAVE + 16,), jnp.int32),
            pltpu.VMEM((_WAVE + 16,), jnp.int32),
            pltpu.VMEM((_WAVE, D), jnp.float32),
            pltpu.VMEM((D, 128), jnp.float32),
            pltpu.SemaphoreType.DMA,
            pltpu.SemaphoreType.DMA,
            pltpu.SemaphoreType.DMA,
        ],
        compiler_params=pltpu.CompilerParams(needs_layout_passes=False),
    )
    def gather_kernel(tab_hbm, tail_hbm, ids_hbm, out_hbm, ids_v, lid_v,
                      lpos_v, slab_a, slab_b, wl_id, wl_pos, wrows, tail_v,
                      sem_a, sem_b, sem_o):
        wid = lax.axis_index("s") * NC + lax.axis_index("c")
        lanes = lax.iota(jnp.int32, 16)
        pltpu.sync_copy(ids_hbm, ids_v)

        def filt(g, off):
            idv = ids_v[pl.ds(g * 16, 16)]
            posv = g * 16 + lanes
            chv = lax.shift_right_logical(idv, 9)
            m = (chv & (NW - 1)) == wid
            cnt = plsc.all_reduce_population_count(m)[0]
            plsc.store_compressed(lid_v.at[pl.ds(off, 16)], idv, mask=m)
            plsc.store_compressed(lpos_v.at[pl.ds(off, 16)], posv, mask=m)
            return off + cnt

        n_loc = lax.fori_loop(0, nvec, filt, 0)
        n_lvec = (n_loc + 15) // 16

        def process_wave(n, cbase, slab):
            def one(j, _):
                j16 = jnp.full((16,), 0, jnp.int32) + j
                id16 = plsc.load_gather(wl_id, [j16])
                pos16 = plsc.load_gather(wl_pos, [j16])
                p16 = id16 - cbase
                for k in range(D // 16):
                    col = plsc.load_gather(slab, [k * 16 + lanes, p16])
                    wrows[j, pl.ds(k * 16, 16)] = col
                pltpu.async_copy(
                    wrows.at[pl.ds(j, 1)],
                    out_hbm.at[pl.ds(pos16[0], 1)],
                    sem_o,
                )
                return 0

            lax.fori_loop(0, n, one, 0)

            def drain(j, _):
                pltpu.make_async_copy(
                    out_hbm.at[pl.ds(0, 1)], wrows.at[pl.ds(0, 1)], sem_o
                ).wait()
                return 0

            lax.fori_loop(0, n, drain, 0)

        def scan_chunk(c, slab):
            cbase = c * _CH

            def body(carry):
                gv, wcnt = carry
                idv = lid_v[pl.ds(gv * 16, 16)]
                posv = lpos_v[pl.ds(gv * 16, 16)]
                inb = (gv * 16 + lanes) < n_loc
                m = (lax.shift_right_logical(idv, 9) == c) & inb
                cnt = plsc.all_reduce_population_count(m)[0]
                plsc.store_compressed(wl_id.at[pl.ds(wcnt, 16)], idv, mask=m)
                plsc.store_compressed(wl_pos.at[pl.ds(wcnt, 16)], posv, mask=m)
                wcnt2 = wcnt + cnt

                def flush(n):
                    process_wave(n, cbase, slab)
                    return 0

                wcnt3 = lax.cond(
                    wcnt2 > _WAVE - 16, flush, lambda n: n, wcnt2
                )
                return gv + 1, wcnt3

            def cond(carry):
                return carry[0] < n_lvec

            _, wrem = lax.while_loop(cond, body, (0, 0))

            @pl.when(wrem > 0)
            def _():
                process_wave(wrem, cbase, slab)

        def issue(c, slab, sem):
            pltpu.async_copy(tab_hbm.at[:, pl.ds(c * _CH, _CH)], slab, sem)

        def wait_slab(slab, sem):
            pltpu.make_async_copy(
                tab_hbm.at[:, pl.ds(0, _CH)], slab, sem
            ).wait()

        @pl.when(wid < n_full)
        def _():
            issue(wid, slab_a, sem_a)

        def step(k, _):
            c = wid + k * NW
            nxt = c + NW

            def even(_):
                @pl.when(nxt < n_full)
                def _():
                    issue(nxt, slab_b, sem_b)

                @pl.when(c < n_full)
                def _():
                    wait_slab(slab_a, sem_a)
                    scan_chunk(c, slab_a)

                return 0

            def odd(_):
                @pl.when(nxt < n_full)
                def _():
                    issue(nxt, slab_a, sem_a)

                @pl.when(c < n_full)
                def _():
                    wait_slab(slab_b, sem_b)
                    scan_chunk(c, slab_b)

                return 0

            lax.cond(k % 2 == 0, even, odd, 0)
            return 0

        lax.fori_loop(0, K, step, 0)

        if tail:
            @pl.when(wid == tail_owner)
            def _():
                pltpu.sync_copy(tail_hbm, tail_v)
                scan_chunk(n_full, tail_v)

    tail_cols = jnp.zeros((D, 128), tableT.dtype)
    tail_cols = tail_cols.at[:, :tail].set(tableT[:, n_full * _CH:])
    return gather_kernel(tableT, tail_cols, ids)


def _tc_combine(uf, tv, w1, w2, b2d):
    B, D = uf.shape
    bB = 2048

    def body(uf_ref, tv_ref, w1_ref, w2_ref, b_ref, o_ref):
        acc = jnp.dot(uf_ref[...], w1_ref[...], preferred_element_type=jnp.float32)
        acc += jnp.dot(tv_ref[...], w2_ref[...], preferred_element_type=jnp.float32)
        o_ref[...] = acc + b_ref[...]

    return pl.pallas_call(
        body,
        grid=(B // bB,),
        in_specs=[
            pl.BlockSpec((bB, D), lambda i: (i, 0)),
            pl.BlockSpec((bB, D), lambda i: (i, 0)),
            pl.BlockSpec((D, D), lambda i: (0, 0)),
            pl.BlockSpec((D, D), lambda i: (0, 0)),
            pl.BlockSpec((1, D), lambda i: (0, 0)),
        ],
        out_specs=pl.BlockSpec((bB, D), lambda i: (i, 0)),
        out_shape=jax.ShapeDtypeStruct((B, D), jnp.float32),
        compiler_params=pltpu.CompilerParams(
            dimension_semantics=("arbitrary",),
        ),
    )(uf, tv, w1, w2, b2d)


@jax.jit
def kernel(content, table, W, b):
    D = table.shape[1]
    ids = content[:, 0].astype(jnp.int32)
    tv = content[:, 1:]
    uf = _sc_gather_scan(table.T, ids)
    w1 = W[:, :D].T
    w2 = W[:, D:].T
    return _tc_combine(uf, tv, w1, w2, b.reshape(1, D))


# speedup vs baseline: 4.2298x; 1.0910x over previous
import functools

import jax
import jax.numpy as jnp
from jax import lax
from jax.experimental import pallas as pl
from jax.experimental.pallas import tpu as pltpu
from jax.experimental.pallas import tpu_sc as plsc

_CH = 512
_WAVE = 32


def _sc_gather_scan(tableT, ids):
    D, V = tableT.shape
    B = ids.shape[0]
    info = plsc.get_sparse_core_info()
    NC, NS = info.num_cores, info.num_subcores
    NW = NC * NS
    n_full = V // _CH
    tail = V - n_full * _CH
    K = (n_full + NW - 1) // NW
    tail_owner = n_full % NW
    nvec = B // 16
    mesh = plsc.VectorSubcoreMesh(core_axis_name="c", subcore_axis_name="s")

    @functools.partial(
        pl.kernel,
        mesh=mesh,
        out_type=jax.ShapeDtypeStruct((B, D), jnp.float32),
        scratch_types=[
            pltpu.VMEM((B,), jnp.int32),
            pltpu.VMEM((B,), jnp.int32),
            pltpu.VMEM((B,), jnp.int32),
            pltpu.VMEM((D, _CH), jnp.float32),
            pltpu.VMEM((D, _CH), jnp.float32),
            pltpu.VMEM((_WAVE + 16,), jnp.int32),
            pltpu.VMEM((_WAVE + 16,), jnp.int32),
            pltpu.VMEM((_WAVE, D), jnp.float32),
            pltpu.VMEM((D, 128), jnp.float32),
            pltpu.SemaphoreType.DMA,
            pltpu.SemaphoreType.DMA,
            pltpu.SemaphoreType.DMA,
        ],
        compiler_params=pltpu.CompilerParams(needs_layout_passes=False),
    )
    def gather_kernel(tab_hbm, tail_hbm, ids_hbm, out_hbm, ids_v, lid_v,
                      lpos_v, slab_a, slab_b, wl_id, wl_pos, wrows, tail_v,
                      sem_a, sem_b, sem_o):
        wid = lax.axis_index("s") * NC + lax.axis_index("c")
        lanes = lax.iota(jnp.int32, 16)

        @pl.when(wid < n_full)
        def _():
            pltpu.async_copy(
                tab_hbm.at[:, pl.ds(wid * _CH, _CH)], slab_a, sem_a
            )

        pltpu.sync_copy(ids_hbm, ids_v)

        def filt(g, off):
            idv = ids_v[pl.ds(g * 16, 16)]
            posv = g * 16 + lanes
            chv = lax.shift_right_logical(idv, 9)
            m = (chv & (NW - 1)) == wid
            cnt = plsc.all_reduce_population_count(m)[0]
            plsc.store_compressed(lid_v.at[pl.ds(off, 16)], idv, mask=m)
            plsc.store_compressed(lpos_v.at[pl.ds(off, 16)], posv, mask=m)
            return off + cnt

        n_loc = lax.fori_loop(0, nvec, filt, 0)
        n_lvec = (n_loc + 15) // 16

        def drain_rows(n):
            def drain(j, _):
                pltpu.make_async_copy(
                    out_hbm.at[pl.ds(0, 1)], wrows.at[pl.ds(0, 1)], sem_o
                ).wait()
                return 0

            lax.fori_loop(0, n, drain, 0)

        def process_wave(n, cbase, slab):
            def one(j, _):
                j16 = jnp.full((16,), 0, jnp.int32) + j
                id16 = plsc.load_gather(wl_id, [j16])
                pos16 = plsc.load_gather(wl_pos, [j16])
                p16 = id16 - cbase
                for k in range(D // 16):
                    col = plsc.load_gather(slab, [k * 16 + lanes, p16])
                    wrows[j, pl.ds(k * 16, 16)] = col
                pltpu.async_copy(
                    wrows.at[pl.ds(j, 1)],
                    out_hbm.at[pl.ds(pos16[0], 1)],
                    sem_o,
                )
                return 0

            lax.fori_loop(0, n, one, 0)

        def scan_chunk(c, slab):
            cbase = c * _CH

            def body(carry):
                gv, wcnt = carry
                idv = lid_v[pl.ds(gv * 16, 16)]
                posv = lpos_v[pl.ds(gv * 16, 16)]
                inb = (gv * 16 + lanes) < n_loc
                m = (lax.shift_right_logical(idv, 9) == c) & inb
                cnt = plsc.all_reduce_population_count(m)[0]
                plsc.store_compressed(wl_id.at[pl.ds(wcnt, 16)], idv, mask=m)
                plsc.store_compressed(wl_pos.at[pl.ds(wcnt, 16)], posv, mask=m)
                wcnt2 = wcnt + cnt

                def flush(n):
                    process_wave(n, cbase, slab)
                    drain_rows(n)
                    return 0

                wcnt3 = lax.cond(
                    wcnt2 > _WAVE - 16, flush, lambda n: n, wcnt2
                )
                return gv + 1, wcnt3

            def cond(carry):
                return carry[0] < n_lvec

            _, wrem = lax.while_loop(cond, body, (0, 0))

            @pl.when(wrem > 0)
            def _():
                process_wave(wrem, cbase, slab)

            return wrem

        def issue(c, slab, sem):
            pltpu.async_copy(tab_hbm.at[:, pl.ds(c * _CH, _CH)], slab, sem)

        def wait_slab(slab, sem):
            pltpu.make_async_copy(
                tab_hbm.at[:, pl.ds(0, _CH)], slab, sem
            ).wait()

        def step(k, pending):
            c = wid + k * NW
            nxt = c + NW

            def run(slab, sem, oslab, osem):
                @pl.when(nxt < n_full)
                def _():
                    issue(nxt, oslab, osem)

                def go(p):
                    wait_slab(slab, sem)
                    drain_rows(p)
                    return scan_chunk(c, slab)

                return lax.cond(c < n_full, go, lambda p: p, pending)

            def even(p):
                return run(slab_a, sem_a, slab_b, sem_b)

            def odd(p):
                return run(slab_b, sem_b, slab_a, sem_a)

            return lax.cond(k % 2 == 0, even, odd, pending)

        pending = lax.fori_loop(0, K, step, 0)

        if tail:
            def tail_go(p):
                pltpu.sync_copy(tail_hbm, tail_v)
                drain_rows(p)
                return scan_chunk(n_full, tail_v)

            pending = lax.cond(wid == tail_owner, tail_go, lambda p: p, pending)

        drain_rows(pending)

    tail_cols = jnp.zeros((D, 128), tableT.dtype)
    tail_cols = tail_cols.at[:, :tail].set(tableT[:, n_full * _CH:])
    return gather_kernel(tableT, tail_cols, ids)


def _tc_combine_t(uf, contentT, w1, w2, bcol):
    B, D = uf.shape
    bB = 2048

    def body(uf_ref, c_ref, w1_ref, w2_ref, b_ref, o_ref):
        acc = lax.dot_general(
            w1_ref[...], uf_ref[...],
            (((1,), (1,)), ((), ())),
            preferred_element_type=jnp.float32,
        )
        acc += jnp.dot(
            w2_ref[...], c_ref[1:, :], preferred_element_type=jnp.float32
        )
        o_ref[...] = acc + b_ref[...]

    return pl.pallas_call(
        body,
        grid=(B // bB,),
        in_specs=[
            pl.BlockSpec((bB, D), lambda i: (i, 0)),
            pl.BlockSpec((D + 1, bB), lambda i: (0, i)),
            pl.BlockSpec((D, D), lambda i: (0, 0)),
            pl.BlockSpec((D, D), lambda i: (0, 0)),
            pl.BlockSpec((D, 1), lambda i: (0, 0)),
        ],
        out_specs=pl.BlockSpec((D, bB), lambda i: (0, i)),
        out_shape=jax.ShapeDtypeStruct((D, B), jnp.float32),
        compiler_params=pltpu.CompilerParams(
            dimension_semantics=("arbitrary",),
        ),
    )(uf, contentT, w1, w2, bcol)


@jax.jit
def kernel(content, table, W, b):
    D = table.shape[1]
    ids = content[:, 0].astype(jnp.int32)
    uf = _sc_gather_scan(table.T, ids)
    w1 = W[:, :D]
    w2 = W[:, D:]
    outT = _tc_combine_t(uf, content.T, w1, w2, b.reshape(D, 1))
    return outT.T
